# Initial kernel scaffold; baseline (speedup 1.0000x reference)
import jax
import jax.numpy as jnp
from jax.experimental import pallas as pl


def kernel(features, edge_index, W, b):
    raise NotImplementedError("write your pallas kernel here")


# speedup vs baseline: 4.2251x; 4.2251x over previous
import functools

import jax
import jax.numpy as jnp
from jax import lax
from jax.experimental import pallas as pl
from jax.experimental.pallas import tpu as pltpu
from jax.experimental.pallas import tpu_sc as plsc

N = 10000
E = 320000
D = 128

NC = 2
NS = 16
NW = NC * NS

NP = 10240
NPT = NP // NS

EPW = E // NW
C = 80
NCHUNK = EPW // C

DEG_W = 8

BR = 512

_mesh = plsc.VectorSubcoreMesh(core_axis_name="c", subcore_axis_name="s")


@functools.partial(
    pl.kernel,
    out_type=jax.ShapeDtypeStruct((NC, NP, DEG_W), jnp.float32),
    mesh=_mesh,
    scratch_types=[
        pltpu.VMEM((C,), jnp.int32),
        pltpu.VMEM((C, DEG_W), jnp.float32),
        pltpu.VMEM_SHARED((NP, DEG_W), jnp.float32),
    ],
)
def _deg_sc(dst_hbm, ones_hbm, zeros_hbm, out_hbm, dst_v, ones_v, acc):
    cid = lax.axis_index("c")
    sid = lax.axis_index("s")
    wid = sid * NC + cid
    pltpu.sync_copy(ones_hbm, ones_v)
    pltpu.sync_copy(zeros_hbm, acc.at[pl.ds(sid * NPT, NPT)])
    plsc.subcore_barrier()

    def body(i, carry):
        base = wid * EPW + i * C
        pltpu.sync_copy(dst_hbm.at[pl.ds(base, C)], dst_v)
        pltpu.sync_copy(ones_v, acc.at[dst_v], add=True)
        return carry

    lax.fori_loop(0, NCHUNK, body, 0)
    plsc.subcore_barrier()
    pltpu.sync_copy(acc.at[pl.ds(sid * NPT, NPT)],
                    out_hbm.at[cid, pl.ds(sid * NPT, NPT)])


@functools.partial(
    pl.kernel,
    out_type=jax.ShapeDtypeStruct((NC, NP, D), jnp.float32),
    mesh=_mesh,
    scratch_types=[
        pltpu.VMEM((C,), jnp.int32),
        pltpu.VMEM((C,), jnp.int32),
        pltpu.VMEM((C, D), jnp.float32),
        pltpu.VMEM_SHARED((NP, D), jnp.float32),
        pltpu.SemaphoreType.DMA,
    ],
)
def _agg_sc(src_hbm, dst_hbm, hs_hbm, zeros_hbm, out_hbm,
            src_v, dst_v, rows_v, acc, sem):
    cid = lax.axis_index("c")
    sid = lax.axis_index("s")
    wid = sid * NC + cid
    pltpu.sync_copy(zeros_hbm, acc.at[pl.ds(sid * NPT, NPT)])
    plsc.subcore_barrier()

    def body(i, carry):
        base = wid * EPW + i * C
        pltpu.sync_copy(src_hbm.at[pl.ds(base, C)], src_v)
        pltpu.sync_copy(dst_hbm.at[pl.ds(base, C)], dst_v)
        pltpu.async_copy(hs_hbm.at[src_v], rows_v, sem).wait()
        pltpu.sync_copy(rows_v, acc.at[dst_v], add=True)
        return carry

    lax.fori_loop(0, NCHUNK, body, 0)
    plsc.subcore_barrier()
    pltpu.sync_copy(acc.at[pl.ds(sid * NPT, NPT)],
                    out_hbm.at[cid, pl.ds(sid * NPT, NPT)])


def _dense_body(x_ref, wt_ref, b_ref, n05_ref, h0_ref, hs0_ref):
    h0 = jnp.dot(x_ref[...], wt_ref[...],
                 preferred_element_type=jnp.float32) + b_ref[0:1, :]
    h0_ref[...] = h0
    hs0_ref[...] = h0 * n05_ref[...]


_dense_tc = pl.pallas_call(
    _dense_body,
    grid=(NP // BR,),
    in_specs=[
        pl.BlockSpec((BR, D), lambda i: (i, 0)),
        pl.BlockSpec((D, D), lambda i: (0, 0)),
        pl.BlockSpec((8, D), lambda i: (0, 0)),
        pl.BlockSpec((BR, D), lambda i: (i, 0)),
    ],
    out_specs=[pl.BlockSpec((BR, D), lambda i: (i, 0)),
               pl.BlockSpec((BR, D), lambda i: (i, 0))],
    out_shape=[jax.ShapeDtypeStruct((NP, D), jnp.float32),
               jax.ShapeDtypeStruct((NP, D), jnp.float32)],
)


def _combine_body(p_ref, h0_ref, va_ref, vb_ref, o_ref):
    o_ref[...] = ((p_ref[0] + p_ref[1]) * va_ref[...]
                  + h0_ref[...] * vb_ref[...])


_combine_tc = pl.pallas_call(
    _combine_body,
    grid=(NP // BR,),
    in_specs=[
        pl.BlockSpec((NC, BR, D), lambda i: (0, i, 0)),
        pl.BlockSpec((BR, D), lambda i: (i, 0)),
        pl.BlockSpec((BR, D), lambda i: (i, 0)),
        pl.BlockSpec((BR, D), lambda i: (i, 0)),
    ],
    out_specs=pl.BlockSpec((BR, D), lambda i: (i, 0)),
    out_shape=jax.ShapeDtypeStruct((NP, D), jnp.float32),
)


def kernel(features, edge_index, W, b):
    src = edge_index[0]
    dst = edge_index[1]
    feat_p = jnp.pad(features, ((0, NP - N), (0, 0)))
    wt = W.T
    b2 = jnp.broadcast_to(b[None, :], (8, D))
    ones_in = jnp.ones((C, DEG_W), jnp.float32)
    zeros_deg = jnp.zeros((NPT, DEG_W), jnp.float32)
    zeros_row = jnp.zeros((NPT, D), jnp.float32)

    deg_p = _deg_sc(dst, ones_in, zeros_deg)
    degs = jnp.sum(deg_p, axis=(0, 2))
    d1 = degs + 1.0
    n05 = lax.rsqrt(d1)
    nl1 = 1.0 / d1
    n05_b = jnp.broadcast_to(n05[:, None], (NP, D))
    nl1_b = jnp.broadcast_to(nl1[:, None], (NP, D))
    nl1n05_b = jnp.broadcast_to((nl1 * n05)[:, None], (NP, D))

    h0, hs0 = _dense_tc(feat_p, wt, b2, n05_b)
    p1 = _agg_sc(src, dst, hs0, zeros_row)
    hs1 = _combine_tc(p1, h0, nl1_b, nl1n05_b)
    p2 = _agg_sc(src, dst, hs1, zeros_row)
    out = _combine_tc(p2, h0, n05_b, nl1_b)
    return out[:N]
